# speedup vs baseline: 1.6687x; 1.6687x over previous
import jax
import jax.numpy as jnp
from jax.experimental import pallas as pl

T_BLK = 1024


def _add_body(x_ref, pe_ref, o_ref):
    o_ref[...] = x_ref[...] + pe_ref[...]


def kernel(x, pe):
    B, T, D = x.shape
    n_t = T // T_BLK
    return pl.pallas_call(
        _add_body,
        grid=(n_t, B),
        in_specs=[
            pl.BlockSpec((None, T_BLK, D), lambda i, b: (b, i, 0)),
            pl.BlockSpec((T_BLK, D), lambda i, b: (i, 0)),
        ],
        out_specs=pl.BlockSpec((None, T_BLK, D), lambda i, b: (b, i, 0)),
        out_shape=jax.ShapeDtypeStruct((B, T, D), x.dtype),
    )(x, pe[:T])
